# baseline (speedup 1.0000x reference)
import jax
import jax.numpy as jnp
from jax import lax
from jax.experimental import pallas as pl
from jax.experimental.pallas import tpu as pltpu
from jax.experimental.pallas import tpu_sc as plsc

_NI, _NJ = 4096, 200
_B = _NI * _NJ
_D = 30
_DP = 32
_NC, _NS = 2, 16
_NW = _NC * _NS
_BPW = _B // _NW
_TJ = _NJ // 8
_WG = 512
_NWG = _BPW // _WG


def _emb_body(x_hbm, table_hbm, out_hbm, idx_all,
              wgi0, wgi1, rows0, rows1, comp0, comp1,
              gsem0, gsem1, wsem0, wsem1):
    wid = lax.axis_index("s") * _NC + lax.axis_index("c")
    base = wid * _BPW
    wgi = (wgi0, wgi1)
    rows = (rows0, rows1)
    comp = (comp0, comp1)
    gsem = (gsem0, gsem1)
    wsem = (wsem0, wsem1)

    iota = lax.iota(jnp.int32, 16)
    i200 = iota * 200

    pltpu.sync_copy(x_hbm.at[pl.ds(base, _BPW)], idx_all)

    def build_idx(c, b):
        j0 = (c // 2) * 8 + (c % 2) * 4
        for v in range(32):
            jj = v // 8
            r = v % 8
            src = i200 + (3200 * r + j0 + jj)
            vals = plsc.load_gather(idx_all, [src])
            wgi[b][pl.ds(16 * v, 16)] = vals

    def g_start(b):
        pltpu.async_copy(table_hbm.at[wgi[b]], rows[b], gsem[b])

    def g_wait(b):
        pltpu.make_async_copy(table_hbm.at[wgi[b]], rows[b], gsem[b]).wait()

    pds = [(iota + d) & 15 for d in range(16)]

    def produce(b):
        for h in (0, 1):
            kbase = 14 * h

            @plsc.parallel_loop(0, _WG // 16, step=1)
            def blk(rb):
                r0 = 16 * rb
                rvec = r0 + iota
                jjv = jnp.full((16,), rb // 8, jnp.int32)
                iiv = 16 * (rb % 8) + iota
                for d in range(16):
                    kv = pds[d] + kbase
                    vals = plsc.load_gather(rows[b], [rvec, kv])
                    plsc.store_scatter(comp[b], [kv, jjv, iiv], vals)

    def w_start(c, b):
        tj = c // 2
        js = c % 2
        pltpu.async_copy(
            comp[b],
            out_hbm.at[:, tj, wid, pl.ds(4 * js, 4), :],
            wsem[b])

    def w_wait(b):
        pltpu.make_async_copy(
            comp[b], out_hbm.at[:, 0, 0, pl.ds(0, 4), :], wsem[b]).wait()

    build_idx(0, 0)
    g_start(0)

    def step(c, b):
        g_wait(b)

        @pl.when(c + 1 < _NWG)
        def _():
            build_idx(c + 1, 1 - b)
            g_start(1 - b)

        @pl.when(c >= 2)
        def _():
            w_wait(b)

        produce(b)
        w_start(c, b)

    def body(p, carry):
        for b in range(2):
            step(2 * p + b, b)
        return carry

    lax.fori_loop(0, _NWG // 2, body, 0)

    w_wait(0)
    w_wait(1)


def kernel(x, table):
    xf = x.reshape(-1)
    tpad = jnp.pad(table, ((0, 0), (0, _DP - _D)))
    mesh = plsc.VectorSubcoreMesh(core_axis_name="c", subcore_axis_name="s")
    f = pl.kernel(
        _emb_body,
        mesh=mesh,
        out_type=jax.ShapeDtypeStruct((_D, _TJ, _NW, 8, 128), jnp.float32),
        scratch_types=[
            pltpu.VMEM((_BPW,), jnp.int32),
            pltpu.VMEM((_WG,), jnp.int32),
            pltpu.VMEM((_WG,), jnp.int32),
            pltpu.VMEM((_WG, _DP), jnp.float32),
            pltpu.VMEM((_WG, _DP), jnp.float32),
            pltpu.VMEM((_D, 4, 128), jnp.float32),
            pltpu.VMEM((_D, 4, 128), jnp.float32),
            pltpu.SemaphoreType.DMA,
            pltpu.SemaphoreType.DMA,
            pltpu.SemaphoreType.DMA,
            pltpu.SemaphoreType.DMA,
        ],
        compiler_params=pltpu.CompilerParams(
            use_tc_tiling_on_sc=False, needs_layout_passes=False,
            disable_bounds_checks=True, disable_semaphore_checks=True,
            skip_device_barrier=True),
    )
    out5 = f(xf, tpad)
    return out5.transpose(2, 4, 1, 3, 0).reshape(_NI, _NJ, _D)


# speedup vs baseline: 1.0352x; 1.0352x over previous
import jax
import jax.numpy as jnp
from jax import lax
from jax.experimental import pallas as pl
from jax.experimental.pallas import tpu as pltpu
from jax.experimental.pallas import tpu_sc as plsc

_NI, _NJ = 4096, 200
_B = _NI * _NJ
_D = 30
_DP = 32
_NC, _NS = 2, 16
_NW = _NC * _NS
_BPW = _B // _NW
_TJ = _NJ // 8
_WG = 512
_NWG = _BPW // _WG


def _emb_body(x_hbm, table_hbm, out_hbm, idx_all,
              wgi0, wgi1, rows0, rows1, comp0, comp1,
              gsem0, gsem1, gsem20, gsem21, wsem0, wsem1):
    wid = lax.axis_index("s") * _NC + lax.axis_index("c")
    base = wid * _BPW
    wgi = (wgi0, wgi1)
    rows = (rows0, rows1)
    comp = (comp0, comp1)
    gsem = (gsem0, gsem1)
    gsem2 = (gsem20, gsem21)
    wsem = (wsem0, wsem1)

    iota = lax.iota(jnp.int32, 16)
    i200 = iota * 200

    pltpu.sync_copy(x_hbm.at[pl.ds(base, _BPW)], idx_all)

    def build_idx(c, b):
        j0 = (c // 2) * 8 + (c % 2) * 4
        for v in range(32):
            jj = v // 8
            r = v % 8
            src = i200 + (3200 * r + j0 + jj)
            vals = plsc.load_gather(idx_all, [src])
            wgi[b][pl.ds(16 * v, 16)] = vals

    def g_start(b):
        h = _WG // 2
        pltpu.async_copy(
            table_hbm.at[wgi[b].at[pl.ds(0, h)]],
            rows[b].at[pl.ds(0, h)], gsem[b])
        pltpu.async_copy(
            table_hbm.at[wgi[b].at[pl.ds(h, h)]],
            rows[b].at[pl.ds(h, h)], gsem2[b])

    def g_wait(b):
        h = _WG // 2
        pltpu.make_async_copy(
            table_hbm.at[wgi[b].at[pl.ds(0, h)]],
            rows[b].at[pl.ds(0, h)], gsem[b]).wait()
        pltpu.make_async_copy(
            table_hbm.at[wgi[b].at[pl.ds(0, h)]],
            rows[b].at[pl.ds(h, h)], gsem2[b]).wait()

    pds = [(iota + d) & 15 for d in range(16)]

    def produce(b):
        for h in (0, 1):
            kbase = 14 * h

            @plsc.parallel_loop(0, _WG // 16, step=1)
            def blk(rb):
                r0 = 16 * rb
                rvec = r0 + iota
                jjv = jnp.full((16,), rb // 8, jnp.int32)
                iiv = 16 * (rb % 8) + iota
                for d in range(16):
                    kv = pds[d] + kbase
                    vals = plsc.load_gather(rows[b], [rvec, kv])
                    plsc.store_scatter(comp[b], [kv, jjv, iiv], vals)

    def w_start(c, b):
        tj = c // 2
        js = c % 2
        pltpu.async_copy(
            comp[b],
            out_hbm.at[:, tj, wid, pl.ds(4 * js, 4), :],
            wsem[b])

    def w_wait(b):
        pltpu.make_async_copy(
            comp[b], out_hbm.at[:, 0, 0, pl.ds(0, 4), :], wsem[b]).wait()

    build_idx(0, 0)
    g_start(0)

    def step(c, b):
        g_wait(b)

        @pl.when(c + 1 < _NWG)
        def _():
            build_idx(c + 1, 1 - b)
            g_start(1 - b)

        @pl.when(c >= 2)
        def _():
            w_wait(b)

        produce(b)
        w_start(c, b)

    def body(p, carry):
        for b in range(2):
            step(2 * p + b, b)
        return carry

    lax.fori_loop(0, _NWG // 2, body, 0)

    w_wait(0)
    w_wait(1)


def kernel(x, table):
    xf = x.reshape(-1)
    tpad = jnp.pad(table, ((0, 0), (0, _DP - _D)))
    mesh = plsc.VectorSubcoreMesh(core_axis_name="c", subcore_axis_name="s")
    f = pl.kernel(
        _emb_body,
        mesh=mesh,
        out_type=jax.ShapeDtypeStruct((_D, _TJ, _NW, 8, 128), jnp.float32),
        scratch_types=[
            pltpu.VMEM((_BPW,), jnp.int32),
            pltpu.VMEM((_WG,), jnp.int32),
            pltpu.VMEM((_WG,), jnp.int32),
            pltpu.VMEM((_WG, _DP), jnp.float32),
            pltpu.VMEM((_WG, _DP), jnp.float32),
            pltpu.VMEM((_D, 4, 128), jnp.float32),
            pltpu.VMEM((_D, 4, 128), jnp.float32),
            pltpu.SemaphoreType.DMA,
            pltpu.SemaphoreType.DMA,
            pltpu.SemaphoreType.DMA,
            pltpu.SemaphoreType.DMA,
            pltpu.SemaphoreType.DMA,
            pltpu.SemaphoreType.DMA,
        ],
        compiler_params=pltpu.CompilerParams(
            use_tc_tiling_on_sc=False, needs_layout_passes=False,
            disable_bounds_checks=True, disable_semaphore_checks=True,
            skip_device_barrier=True),
    )
    out5 = f(xf, tpad)
    return out5.transpose(2, 4, 1, 3, 0).reshape(_NI, _NJ, _D)
